# baseline (speedup 1.0000x reference)
import functools
import math

import jax
import jax.numpy as jnp
from jax import lax
from jax.experimental import pallas as pl
from jax.experimental.pallas import tpu as pltpu
from jax.experimental.pallas import tpu_sc as plsc

_NC = 2
_NS = 16
_NW = _NC * _NS
_LANES = 16
_NH = 4


def _build_sc_lookup(seq, batch, d):
    ppw = seq // _NW
    hp = ppw // _NH
    scale = float(math.sqrt(d))
    mesh = plsc.VectorSubcoreMesh(core_axis_name="c", subcore_axis_name="s")

    @functools.partial(
        pl.kernel,
        mesh=mesh,
        out_type=jax.ShapeDtypeStruct((seq, batch, d), jnp.float32),
        scratch_types=[
            pltpu.VMEM((batch, ppw), jnp.int32),
            pltpu.VMEM((batch, hp, d), jnp.float32),
            pltpu.VMEM((batch, hp, d), jnp.float32),
            pltpu.VMEM((hp, batch, d), jnp.float32),
            pltpu.VMEM((hp, batch, d), jnp.float32),
            pltpu.VMEM((hp, d), jnp.float32),
            pltpu.VMEM((hp, d), jnp.float32),
            pltpu.SemaphoreType.DMA,
            pltpu.SemaphoreType.DMA,
            pltpu.SemaphoreType.DMA,
            pltpu.SemaphoreType.DMA,
        ],
    )
    def sc_lookup(table_hbm, xt_hbm, pos_hbm, out_hbm, idxb_v,
                  g0, g1, o0, o1, p0, p1, gs0, gs1, os0, os1):
        wid = lax.axis_index("s") * _NC + lax.axis_index("c")
        base = wid * ppw
        gbuf, obuf, pbuf = (g0, g1), (o0, o1), (p0, p1)
        gsem, osem = (gs0, gs1), (os0, os1)

        pltpu.sync_copy(xt_hbm.at[:, pl.ds(base, ppw)], idxb_v)

        def fire(h, u):
            cps = [
                pltpu.async_copy(
                    table_hbm.at[idxb_v.at[b, pl.ds(h * hp, hp)]],
                    gbuf[u].at[b],
                    gsem[u],
                )
                for b in range(batch)
            ]
            cps.append(
                pltpu.async_copy(
                    pos_hbm.at[pl.ds(base + h * hp, hp)], pbuf[u], gsem[u]
                )
            )
            return cps

        in_flight = {0: fire(0, 0)}
        out_flight = {}
        for h in range(_NH):
            u = h % 2
            if h + 1 < _NH:
                in_flight[h + 1] = fire(h + 1, 1 - u)
            for cp in in_flight.pop(h):
                cp.wait()
            if h >= 2:
                out_flight.pop(h - 2).wait()

            def step(p, carry, u=u):
                pos_regs = [
                    pbuf[u][p, pl.ds(k * _LANES, _LANES)]
                    for k in range(d // _LANES)
                ]
                for b in range(batch):
                    for k in range(d // _LANES):
                        sl = pl.ds(k * _LANES, _LANES)
                        obuf[u][p, b, sl] = gbuf[u][b, p, sl] * scale + pos_regs[k]
                return carry

            lax.fori_loop(0, hp, step, 0)
            out_flight[h] = pltpu.async_copy(
                obuf[u], out_hbm.at[pl.ds(base + h * hp, hp)], osem[u]
            )
        for h in sorted(out_flight):
            out_flight.pop(h).wait()

    return sc_lookup


def kernel(x, table, pos_emb):
    seq, batch = x.shape
    d = table.shape[1]
    xt = x.T
    pos2 = pos_emb[:seq].reshape(seq, d)
    return _build_sc_lookup(seq, batch, d)(table, xt, pos2)
